# baseline (speedup 1.0000x reference)
import jax
import jax.numpy as jnp
from jax.experimental import pallas as pl
from jax.experimental.pallas import tpu as pltpu

N = 1000
NP = 1024
F = 64
H = 64
EPS = 1e-5
NEG = 0.01
AL = 0.5 * (1.0 + NEG)
BE = 0.5 * (1.0 - NEG)
NPAIR = N * (N - 1) // 2
TB = 128
IB = 8
NIB = N // IB
NJB = NP // TB


def _leaky(x):
    return jnp.where(x >= 0, x, NEG * x)


def _bn_rows(h, g, b):
    mean = jnp.mean(h[0:N, :], axis=0, keepdims=True)
    var = jnp.mean((h[0:N, :] - mean) ** 2, axis=0, keepdims=True)
    return (h - mean) * jax.lax.rsqrt(var + EPS) * g + b


def _prep_body(nf_ref, w1_ref, w2_ref, w3_ref, w5a_ref, w5b_ref, vec_ref,
               a_ref, at_ref, bpt_ref):
    b1 = vec_ref[0:1, :]
    g1 = vec_ref[1:2, :]
    be1 = vec_ref[2:3, :]
    b2 = vec_ref[3:4, :]
    g2 = vec_ref[4:5, :]
    be2 = vec_ref[5:6, :]
    b3 = vec_ref[6:7, :]
    b5 = vec_ref[7:8, :]
    nf = nf_ref[...]
    h = _leaky(jnp.dot(nf, w1_ref[...], preferred_element_type=jnp.float32) + b1)
    h = _bn_rows(h, g1, be1)
    h = _leaky(jnp.dot(h, w2_ref[...], preferred_element_type=jnp.float32) + b2)
    h = _bn_rows(h, g2, be2)
    h = jnp.dot(h, w3_ref[...], preferred_element_type=jnp.float32) + b3 + nf
    a = jnp.dot(h, w5a_ref[...], preferred_element_type=jnp.float32)
    bp = jnp.dot(h, w5b_ref[...], preferred_element_type=jnp.float32) + b5
    a_ref[...] = a
    at_ref[...] = a.T
    bpt_ref[...] = bp.T


def _stats_body(a_ref, bpt_ref, sq_ref, sacc_ref, qacc_ref):
    bi = pl.program_id(0)
    bjb = pl.program_id(1)

    @pl.when((bi == 0) & (bjb == 0))
    def _():
        sacc_ref[...] = jnp.zeros((H, TB), jnp.float32)
        qacc_ref[...] = jnp.zeros((H, TB), jnp.float32)

    jg = bjb * TB + jax.lax.broadcasted_iota(jnp.int32, (1, TB), 1)

    ones_row = jnp.ones((1, TB), jnp.float32)
    c0 = (((0,), (0,)), ((), ()))

    def accumulate(masked):
        bpt = bpt_ref[...]
        s_loc = jnp.zeros((H, TB), jnp.float32)
        q_loc = jnp.zeros((H, TB), jnp.float32)
        for i in range(TB):
            gi = bi * TB + i
            xa = jax.lax.dot_general(a_ref[i:i + 1, :], ones_row, c0,
                                     preferred_element_type=jnp.float32)
            x = xa + bpt
            u = AL * x + BE * jnp.abs(x)
            if masked:
                m = ((jg > gi) & (jg < N) & (gi < N)).astype(jnp.float32)
                u = u * m
            s_loc = s_loc + u
            q_loc = q_loc + u * u
        sacc_ref[...] = sacc_ref[...] + s_loc
        qacc_ref[...] = qacc_ref[...] + q_loc

    active = bjb >= bi
    full = (bjb > bi) & ((bjb + 1) * TB <= N)

    @pl.when(active & full)
    def _():
        accumulate(False)

    @pl.when(active & jnp.logical_not(full))
    def _():
        accumulate(True)

    @pl.when((bi == NJB - 1) & (bjb == NJB - 1))
    def _():
        ssum = jnp.sum(sacc_ref[...], axis=1, keepdims=True)
        qsum = jnp.sum(qacc_ref[...], axis=1, keepdims=True)
        sq_ref[...] = jnp.concatenate(
            [ssum, qsum, jnp.zeros((H, 6), jnp.float32)], axis=1)


def _edge_body(stats_ref, par_ref, atr_ref, bptr_ref, atc_ref, bptc_ref,
               o_ref):
    bi = pl.program_id(0)
    bj = pl.program_id(1)
    inv = 1.0 / NPAIR
    mean = stats_ref[:, 0:1] * inv
    msq = stats_ref[:, 1:2] * inv
    var = msq - mean * mean
    g5 = par_ref[:, 0:1]
    be5 = par_ref[:, 1:2]
    w6d = par_ref[:, 2:3]
    b6d = par_ref[0:1, 3:4]
    scale = g5 * jax.lax.rsqrt(var + EPS)
    wv = scale * w6d
    shift = be5 - mean * scale
    c = jnp.sum(shift * w6d, axis=0, keepdims=True) + b6d
    wa = AL * wv
    wb = BE * wv
    ones_row = jnp.ones((1, TB), jnp.float32)
    c0 = (((0,), (0,)), ((), ()))

    def f(rt_ref, ct_ref):
        sa = jax.lax.dot_general(wa, rt_ref[...], c0,
                                 preferred_element_type=jnp.float32)
        sb = jax.lax.dot_general(wa, ct_ref[...], c0,
                                 preferred_element_type=jnp.float32) + c
        a0 = (jax.lax.dot_general(sa, ones_row, c0,
                                  preferred_element_type=jnp.float32) +
              jax.lax.dot_general(ones_row, sb, c0,
                                  preferred_element_type=jnp.float32))
        a1 = jnp.zeros((TB, TB), jnp.float32)
        a2 = jnp.zeros((TB, TB), jnp.float32)
        a3 = jnp.zeros((TB, TB), jnp.float32)
        accs = [a0, a1, a2, a3]
        for h in range(H):
            o1 = jax.lax.dot_general(rt_ref[h:h + 1, :], ones_row, c0,
                                     preferred_element_type=jnp.float32)
            o2 = jax.lax.dot_general(ones_row, ct_ref[h:h + 1, :], c0,
                                     preferred_element_type=jnp.float32)
            x = o1 + o2
            accs[h % 4] = accs[h % 4] + wb[h:h + 1, 0:1] * jnp.abs(x)
        return (accs[0] + accs[1]) + (accs[2] + accs[3])

    def emit(p0, p1):
        o_ref[...] = jnp.stack([p0, p1], axis=-1).reshape(TB, 2 * TB)

    @pl.when(bi < bj)
    def _():
        d = f(atr_ref, bptc_ref)
        p0 = 1.0 / (1.0 + jnp.exp(-d))
        emit(p0, 1.0 - p0)

    @pl.when(bi > bj)
    def _():
        d = f(bptr_ref, atc_ref)
        p0 = 1.0 / (1.0 + jnp.exp(-d))
        emit(p0, 1.0 - p0)

    @pl.when(bi == bj)
    def _():
        dup = f(atr_ref, bptc_ref)
        dlo = f(bptr_ref, atc_ref)
        rio = jax.lax.broadcasted_iota(jnp.int32, (TB, TB), 0)
        cio = jax.lax.broadcasted_iota(jnp.int32, (TB, TB), 1)
        d = jnp.where(rio < cio, dup, dlo)
        keep = (rio != cio).astype(jnp.float32)
        p0 = keep / (1.0 + jnp.exp(-d))
        emit(p0, keep - p0)


def kernel(x, node_features, w1, b1, g1, be1, w2, b2, g2, be2, w3, b3,
           w5, b5, g5, be5, w6, b6):
    del x
    f32 = jnp.float32
    nf_p = jnp.zeros((NP, F), f32).at[0:N, :].set(node_features)
    w5a = w5[:F, :]
    w5b = w5[F:, :]
    vecs = jnp.stack([b1, g1, be1, b2, g2, be2, b3, b5]).astype(f32)

    a, at, bpt = pl.pallas_call(
        _prep_body,
        out_shape=(
            jax.ShapeDtypeStruct((NP, H), f32),
            jax.ShapeDtypeStruct((H, NP), f32),
            jax.ShapeDtypeStruct((H, NP), f32),
        ),
    )(nf_p, w1, w2, w3, w5a, w5b, vecs)

    stats = pl.pallas_call(
        _stats_body,
        grid=(NJB, NJB),
        in_specs=[
            pl.BlockSpec((TB, H), lambda i, j: (i, 0)),
            pl.BlockSpec((H, TB), lambda i, j: (0, j)),
        ],
        out_specs=pl.BlockSpec((H, 8), lambda i, j: (0, 0)),
        out_shape=jax.ShapeDtypeStruct((H, 8), f32),
        scratch_shapes=[
            pltpu.VMEM((H, TB), f32),
            pltpu.VMEM((H, TB), f32),
        ],
    )(a, bpt)

    w6d = w6[:, 0] - w6[:, 1]
    b6d = jnp.full((H,), b6[0] - b6[1], f32)
    zcol = jnp.zeros((H,), f32)
    params = jnp.stack([g5, be5, w6d, b6d, zcol, zcol, zcol, zcol], axis=1)

    o = pl.pallas_call(
        _edge_body,
        grid=(NP // TB, NP // TB),
        in_specs=[
            pl.BlockSpec((H, 8), lambda i, j: (0, 0)),
            pl.BlockSpec((H, 8), lambda i, j: (0, 0)),
            pl.BlockSpec((H, TB), lambda i, j: (0, i)),
            pl.BlockSpec((H, TB), lambda i, j: (0, i)),
            pl.BlockSpec((H, TB), lambda i, j: (0, j)),
            pl.BlockSpec((H, TB), lambda i, j: (0, j)),
        ],
        out_specs=pl.BlockSpec((TB, 2 * TB), lambda i, j: (i, j)),
        out_shape=jax.ShapeDtypeStruct((NP, 2 * NP), f32),
    )(stats, params, at, bpt, at, bpt)

    return o[:N, :2 * N].reshape(N, N, 2)


# speedup vs baseline: 3.3465x; 3.3465x over previous
import jax
import jax.numpy as jnp
from jax.experimental import pallas as pl
from jax.experimental.pallas import tpu as pltpu

N = 1000
NP = 1024
F = 64
H = 64
EPS = 1e-5
NEG = 0.01
AL = 0.5 * (1.0 + NEG)
BE = 0.5 * (1.0 - NEG)
NPAIR = N * (N - 1) // 2
TB = 128
IB = 8
NIB = N // IB
NJB = NP // TB


def _leaky(x):
    return jnp.where(x >= 0, x, NEG * x)


def _bn_rows(h, g, b):
    mean = jnp.mean(h[0:N, :], axis=0, keepdims=True)
    var = jnp.mean((h[0:N, :] - mean) ** 2, axis=0, keepdims=True)
    return (h - mean) * jax.lax.rsqrt(var + EPS) * g + b


def _prep_body(nf_ref, w1_ref, w2_ref, w3_ref, w5a_ref, w5b_ref, vec_ref,
               a_ref, at_ref, bpt_ref):
    b1 = vec_ref[0:1, :]
    g1 = vec_ref[1:2, :]
    be1 = vec_ref[2:3, :]
    b2 = vec_ref[3:4, :]
    g2 = vec_ref[4:5, :]
    be2 = vec_ref[5:6, :]
    b3 = vec_ref[6:7, :]
    b5 = vec_ref[7:8, :]
    nf = nf_ref[...]
    h = _leaky(jnp.dot(nf, w1_ref[...], preferred_element_type=jnp.float32) + b1)
    h = _bn_rows(h, g1, be1)
    h = _leaky(jnp.dot(h, w2_ref[...], preferred_element_type=jnp.float32) + b2)
    h = _bn_rows(h, g2, be2)
    h = jnp.dot(h, w3_ref[...], preferred_element_type=jnp.float32) + b3 + nf
    a = jnp.dot(h, w5a_ref[...], preferred_element_type=jnp.float32)
    bp = jnp.dot(h, w5b_ref[...], preferred_element_type=jnp.float32) + b5
    a_ref[...] = a
    at_ref[...] = a.T
    bpt_ref[...] = bp.T


def _stats_body(a_ref, bpt_ref, sq_ref, sacc_ref, qacc_ref):
    bi = pl.program_id(0)
    bjb = pl.program_id(1)

    @pl.when((bi == 0) & (bjb == 0))
    def _():
        sacc_ref[...] = jnp.zeros((H, TB), jnp.float32)
        qacc_ref[...] = jnp.zeros((H, TB), jnp.float32)

    jg = bjb * TB + jax.lax.broadcasted_iota(jnp.int32, (1, TB), 1)

    ones_row = jnp.ones((1, TB), jnp.float32)
    c0 = (((0,), (0,)), ((), ()))

    def accumulate(masked):
        bpt = bpt_ref[...]
        s_loc = jnp.zeros((H, TB), jnp.float32)
        q_loc = jnp.zeros((H, TB), jnp.float32)
        for i in range(TB):
            gi = bi * TB + i
            xa = jax.lax.dot_general(a_ref[i:i + 1, :], ones_row, c0,
                                     preferred_element_type=jnp.float32)
            x = xa + bpt
            u = AL * x + BE * jnp.abs(x)
            if masked:
                m = ((jg > gi) & (jg < N) & (gi < N)).astype(jnp.float32)
                u = u * m
            s_loc = s_loc + u
            q_loc = q_loc + u * u
        sacc_ref[...] = sacc_ref[...] + s_loc
        qacc_ref[...] = qacc_ref[...] + q_loc

    active = bjb >= bi
    full = (bjb > bi) & ((bjb + 1) * TB <= N)

    @pl.when(active & full)
    def _():
        accumulate(False)

    @pl.when(active & jnp.logical_not(full))
    def _():
        accumulate(True)

    @pl.when((bi == NJB - 1) & (bjb == NJB - 1))
    def _():
        ssum = jnp.sum(sacc_ref[...], axis=1, keepdims=True)
        qsum = jnp.sum(qacc_ref[...], axis=1, keepdims=True)
        sq_ref[...] = jnp.concatenate(
            [ssum, qsum, jnp.zeros((H, 6), jnp.float32)], axis=1)


def _edge_body(stats_ref, par_ref, atr_ref, bptr_ref, atc_ref, bptc_ref,
               p0_ref, p1_ref):
    bi = pl.program_id(0)
    bj = pl.program_id(1)
    inv = 1.0 / NPAIR
    mean = stats_ref[:, 0:1] * inv
    msq = stats_ref[:, 1:2] * inv
    var = msq - mean * mean
    g5 = par_ref[:, 0:1]
    be5 = par_ref[:, 1:2]
    w6d = par_ref[:, 2:3]
    b6d = par_ref[0:1, 3:4]
    scale = g5 * jax.lax.rsqrt(var + EPS)
    wv = scale * w6d
    shift = be5 - mean * scale
    c = jnp.sum(shift * w6d, axis=0, keepdims=True) + b6d
    wa = AL * wv
    wb = BE * wv
    ones_row = jnp.ones((1, TB), jnp.float32)
    c0 = (((0,), (0,)), ((), ()))

    def f(rt_ref, ct_ref):
        a0 = jnp.zeros((TB, TB), jnp.float32)
        a1 = jnp.zeros((TB, TB), jnp.float32)
        a2 = jnp.zeros((TB, TB), jnp.float32)
        a3 = jnp.zeros((TB, TB), jnp.float32)
        accs = [a0, a1, a2, a3]
        for h in range(H):
            o1 = jax.lax.dot_general(rt_ref[h:h + 1, :], ones_row, c0,
                                     preferred_element_type=jnp.float32)
            o2 = jax.lax.dot_general(ones_row, ct_ref[h:h + 1, :], c0,
                                     preferred_element_type=jnp.float32)
            x = o1 + o2
            accs[h % 4] = (accs[h % 4] + wa[h:h + 1, 0:1] * x +
                           wb[h:h + 1, 0:1] * jnp.abs(x))
        return (accs[0] + accs[1]) + (accs[2] + accs[3]) + c

    @pl.when(bi < bj)
    def _():
        d = f(atr_ref, bptc_ref)
        p0 = 1.0 / (1.0 + jnp.exp(-d))
        p0_ref[...] = p0
        p1_ref[...] = 1.0 - p0

    @pl.when(bi > bj)
    def _():
        d = f(bptr_ref, atc_ref)
        p0 = 1.0 / (1.0 + jnp.exp(-d))
        p0_ref[...] = p0
        p1_ref[...] = 1.0 - p0

    @pl.when(bi == bj)
    def _():
        dup = f(atr_ref, bptc_ref)
        dlo = f(bptr_ref, atc_ref)
        rio = jax.lax.broadcasted_iota(jnp.int32, (TB, TB), 0)
        cio = jax.lax.broadcasted_iota(jnp.int32, (TB, TB), 1)
        d = jnp.where(rio < cio, dup, dlo)
        keep = (rio != cio).astype(jnp.float32)
        p0 = keep / (1.0 + jnp.exp(-d))
        p0_ref[...] = p0
        p1_ref[...] = keep - p0


def kernel(x, node_features, w1, b1, g1, be1, w2, b2, g2, be2, w3, b3,
           w5, b5, g5, be5, w6, b6):
    del x
    f32 = jnp.float32
    nf_p = jnp.zeros((NP, F), f32).at[0:N, :].set(node_features)
    w5a = w5[:F, :]
    w5b = w5[F:, :]
    vecs = jnp.stack([b1, g1, be1, b2, g2, be2, b3, b5]).astype(f32)

    a, at, bpt = pl.pallas_call(
        _prep_body,
        out_shape=(
            jax.ShapeDtypeStruct((NP, H), f32),
            jax.ShapeDtypeStruct((H, NP), f32),
            jax.ShapeDtypeStruct((H, NP), f32),
        ),
    )(nf_p, w1, w2, w3, w5a, w5b, vecs)

    stats = pl.pallas_call(
        _stats_body,
        grid=(NJB, NJB),
        in_specs=[
            pl.BlockSpec((TB, H), lambda i, j: (i, 0)),
            pl.BlockSpec((H, TB), lambda i, j: (0, j)),
        ],
        out_specs=pl.BlockSpec((H, 8), lambda i, j: (0, 0)),
        out_shape=jax.ShapeDtypeStruct((H, 8), f32),
        scratch_shapes=[
            pltpu.VMEM((H, TB), f32),
            pltpu.VMEM((H, TB), f32),
        ],
    )(a, bpt)

    w6d = w6[:, 0] - w6[:, 1]
    b6d = jnp.full((H,), b6[0] - b6[1], f32)
    zcol = jnp.zeros((H,), f32)
    params = jnp.stack([g5, be5, w6d, b6d, zcol, zcol, zcol, zcol], axis=1)

    o = pl.pallas_call(
        _edge_body,
        grid=(NP // TB, NP // TB),
        in_specs=[
            pl.BlockSpec((H, 8), lambda i, j: (0, 0)),
            pl.BlockSpec((H, 8), lambda i, j: (0, 0)),
            pl.BlockSpec((H, TB), lambda i, j: (0, i)),
            pl.BlockSpec((H, TB), lambda i, j: (0, i)),
            pl.BlockSpec((H, TB), lambda i, j: (0, j)),
            pl.BlockSpec((H, TB), lambda i, j: (0, j)),
        ],
        out_specs=[
            pl.BlockSpec((TB, TB), lambda i, j: (i, j)),
            pl.BlockSpec((TB, TB), lambda i, j: (i, j)),
        ],
        out_shape=(
            jax.ShapeDtypeStruct((NP, NP), f32),
            jax.ShapeDtypeStruct((NP, NP), f32),
        ),
    )(stats, params, at, bpt, at, bpt)

    return jnp.stack([o[0][:N, :N], o[1][:N, :N]], axis=-1)
